# baseline (speedup 1.0000x reference)
import functools

import jax
import jax.numpy as jnp
from jax.experimental import pallas as pl
from jax.experimental.pallas import tpu as pltpu

R = 1
K_SEQ = 27
DIAM = 3
ATTN = 64
GEO = 32
SEM = 32
NCLS = 13
M = 20000


def _dense_body(geo_ref, sem_ref, Wg_ref, g_ref, b_ref, Wb_ref, bb_ref,
                Wq_ref, Wk_ref, Wv_ref,
                qgeo_ref, bdy_ref, qp_ref, kw_ref, vw_ref):
    x = geo_ref[...] @ Wg_ref[...]
    mu = jnp.mean(x, axis=-1, keepdims=True)
    var = jnp.mean((x - mu) ** 2, axis=-1, keepdims=True)
    q = (x - mu) / jnp.sqrt(var + 1e-5) * g_ref[...] + b_ref[...]
    q = jnp.maximum(q, 0.0)
    qgeo_ref[...] = q
    bdy_ref[...] = q @ Wb_ref[...] + bb_ref[...]
    qp_ref[...] = q @ Wq_ref[...]
    kw_ref[...] = q @ Wk_ref[...]
    vw_ref[...] = sem_ref[...] @ Wv_ref[...]


def _dense_precompute(geo, sem, W_geo, ln_g, ln_b, W_bdy, b_bdy, Wq, Wk, Wv):
    B = 2000
    grid = (M // B,)
    bs_row = lambda d: pl.BlockSpec((B, d), lambda i: (i, 0))
    bs_full = lambda a, b: pl.BlockSpec((a, b), lambda i: (0, 0))
    out_shapes = (
        jax.ShapeDtypeStruct((M, ATTN), jnp.float32),
        jax.ShapeDtypeStruct((M, 1), jnp.float32),
        jax.ShapeDtypeStruct((M, ATTN), jnp.float32),
        jax.ShapeDtypeStruct((M, ATTN), jnp.float32),
        jax.ShapeDtypeStruct((M, ATTN), jnp.float32),
    )
    return pl.pallas_call(
        _dense_body,
        grid=grid,
        in_specs=[
            bs_row(GEO), bs_row(SEM),
            bs_full(GEO, ATTN), bs_full(1, ATTN), bs_full(1, ATTN),
            bs_full(ATTN, 1), bs_full(1, 1),
            bs_full(ATTN, ATTN), bs_full(ATTN, ATTN), bs_full(SEM, ATTN),
        ],
        out_specs=tuple(bs_row(d) for d in (ATTN, 1, ATTN, ATTN, ATTN)),
        out_shape=out_shapes,
    )(geo, sem, W_geo, ln_g.reshape(1, ATTN), ln_b.reshape(1, ATTN),
      W_bdy, b_bdy.reshape(1, 1), Wq, Wk, Wv)


NPAD = 20480
LEVELS = 5
_IMAX = 2147483647


def _knn_body(q8_ref, c8t_ref, qc2_ref, cc2_ref, out_ref):
    B = q8_ref.shape[0]
    s = jnp.dot(q8_ref[...], c8t_ref[...], preferred_element_type=jnp.int32)
    d = qc2_ref[...] + (cc2_ref[...] - 2 * s)
    col = jax.lax.broadcasted_iota(jnp.int32, (B, NPAD), 1)
    imin = jnp.int32(-2147483648)
    rkeys = d * 32768 + col + imin
    nslice = NPAD // 128

    levels = []
    p_lane = jnp.zeros((B, 128), jnp.int32)
    for lv in range(LEVELS):
        if lv == 0:
            acc = rkeys[:, 0:128]
            for k in range(1, nslice):
                acc = jnp.minimum(acc, rkeys[:, k * 128:(k + 1) * 128])
        else:
            acc = rkeys[:, 0:128] - p_lane
            for k in range(1, nslice):
                acc = jnp.minimum(acc, rkeys[:, k * 128:(k + 1) * 128] - p_lane)
        levels.append(acc + p_lane)
        p_lane = acc + imin + p_lane + 1
    cand = jnp.concatenate(levels, axis=1)

    picked = []
    prev1 = jnp.zeros((B, 1), jnp.int32)
    for _ in range(K_SEQ):
        w = jnp.min(cand - prev1, axis=1, keepdims=True)
        picked.append(w + imin + prev1)
        prev1 = w + imin + prev1 + 1
    fast = jnp.concatenate(picked, axis=1)

    k27_rot = picked[-1] + imin
    suspect = jnp.any(levels[-1] <= k27_rot)

    @pl.when(jnp.logical_not(suspect))
    def _():
        out_ref[...] = fast

    @pl.when(suspect)
    def _():
        slow = []
        p1 = jnp.zeros((B, 1), jnp.int32)
        for _ in range(K_SEQ):
            w = jnp.min(rkeys - p1, axis=1, keepdims=True)
            slow.append(w + imin + p1)
            p1 = w + imin + p1 + 1
        out_ref[...] = jnp.concatenate(slow, axis=1)


def _knn_pallas(coords):
    B = 160
    q8 = coords.astype(jnp.int8)
    c8t = jnp.concatenate([q8.T, jnp.zeros((3, NPAD - M), jnp.int8)], axis=1)
    c2 = jnp.sum(coords * coords, axis=1)
    cc2 = jnp.concatenate([c2, jnp.full((NPAD - M,), 30000, jnp.int32)])
    keys27 = pl.pallas_call(
        _knn_body,
        grid=(M // B,),
        in_specs=[
            pl.BlockSpec((B, 3), lambda i: (i, 0)),
            pl.BlockSpec((3, NPAD), lambda i: (0, 0)),
            pl.BlockSpec((B, 1), lambda i: (i, 0)),
            pl.BlockSpec((1, NPAD), lambda i: (0, 0)),
        ],
        out_specs=pl.BlockSpec((B, K_SEQ), lambda i: (i, 0)),
        out_shape=jax.ShapeDtypeStruct((M, K_SEQ), jnp.int32),
    )(q8, c8t, c2[:, None], cc2[None, :])
    return keys27


def kernel(sp_structure, geo_feat_M, sem_feat_M, W_geo, ln_g, ln_b, W_bdy,
           b_bdy, Wq, Wk, Wv, pos_emb, W_out, b_out, W_cls, b_cls):
    Q_geo, bdy_logits, Q_proj, KW, VW = _dense_precompute(
        geo_feat_M, sem_feat_M, W_geo, ln_g, ln_b, W_bdy, b_bdy, Wq, Wk, Wv)

    coords = sp_structure[:, 1:]
    keys27 = _knn_pallas(coords)
    neighbor_idx = keys27 & 32767
    ndist = keys27 >> 15
    valid_mask = ndist <= 3

    tbl = jnp.concatenate([KW, VW, coords.astype(jnp.float32),
                           jnp.zeros((M, 1), jnp.float32)], axis=1)
    g = jnp.take(tbl, neighbor_idx.reshape(-1), axis=0, mode='clip')
    g = g.reshape(M, K_SEQ, 132)
    K_g = g[..., :ATTN]
    V_proj = g[..., ATTN:2 * ATTN]
    nc = g[..., 2 * ATTN:2 * ATTN + 3].astype(jnp.int32)
    rel = nc - coords[:, None, :]
    rel_int = jnp.clip(rel + R, 0, 2 * R)
    pos_indices = rel_int[:, :, 0] * DIAM ** 2 + rel_int[:, :, 1] * DIAM + rel_int[:, :, 2]

    qpos = Q_proj @ pos_emb.T
    ph = jax.nn.one_hot(pos_indices, DIAM ** 3, dtype=jnp.float32)
    pos_logit = jnp.einsum('mkp,mp->mk', ph, qpos)
    attn_logits = (jnp.einsum('md,mkd->mk', Q_proj, K_g) + pos_logit) / (ATTN ** 0.5)
    attn_logits = jnp.where(valid_mask, attn_logits, -10000.0)
    affinity = jax.nn.softmax(attn_logits, axis=-1)
    refined = jnp.einsum('mk,mkd->md', affinity, V_proj)
    refined = refined + VW
    refined_feat = refined @ W_out + b_out
    logits = refined_feat @ W_cls + b_cls
    return (logits, bdy_logits, affinity[:, None, :], refined_feat,
            neighbor_idx, valid_mask)


# speedup vs baseline: 1.4562x; 1.4562x over previous
import functools

import jax
import jax.numpy as jnp
from jax.experimental import pallas as pl
from jax.experimental.pallas import tpu as pltpu

R = 1
K_SEQ = 27
DIAM = 3
ATTN = 64
GEO = 32
SEM = 32
NCLS = 13
M = 20000


def _dense_body(geo_ref, sem_ref, Wg_ref, g_ref, b_ref, Wb_ref, bb_ref,
                Wq_ref, Wk_ref, Wv_ref,
                qgeo_ref, bdy_ref, qp_ref, kw_ref, vw_ref):
    x = geo_ref[...] @ Wg_ref[...]
    mu = jnp.mean(x, axis=-1, keepdims=True)
    var = jnp.mean((x - mu) ** 2, axis=-1, keepdims=True)
    q = (x - mu) / jnp.sqrt(var + 1e-5) * g_ref[...] + b_ref[...]
    q = jnp.maximum(q, 0.0)
    qgeo_ref[...] = q
    bdy_ref[...] = q @ Wb_ref[...] + bb_ref[...]
    qp_ref[...] = q @ Wq_ref[...]
    kw_ref[...] = q @ Wk_ref[...]
    vw_ref[...] = sem_ref[...] @ Wv_ref[...]


def _dense_precompute(geo, sem, W_geo, ln_g, ln_b, W_bdy, b_bdy, Wq, Wk, Wv):
    B = 2000
    grid = (M // B,)
    bs_row = lambda d: pl.BlockSpec((B, d), lambda i: (i, 0))
    bs_full = lambda a, b: pl.BlockSpec((a, b), lambda i: (0, 0))
    out_shapes = (
        jax.ShapeDtypeStruct((M, ATTN), jnp.float32),
        jax.ShapeDtypeStruct((M, 1), jnp.float32),
        jax.ShapeDtypeStruct((M, ATTN), jnp.float32),
        jax.ShapeDtypeStruct((M, ATTN), jnp.float32),
        jax.ShapeDtypeStruct((M, ATTN), jnp.float32),
    )
    return pl.pallas_call(
        _dense_body,
        grid=grid,
        in_specs=[
            bs_row(GEO), bs_row(SEM),
            bs_full(GEO, ATTN), bs_full(1, ATTN), bs_full(1, ATTN),
            bs_full(ATTN, 1), bs_full(1, 1),
            bs_full(ATTN, ATTN), bs_full(ATTN, ATTN), bs_full(SEM, ATTN),
        ],
        out_specs=tuple(bs_row(d) for d in (ATTN, 1, ATTN, ATTN, ATTN)),
        out_shape=out_shapes,
    )(geo, sem, W_geo, ln_g.reshape(1, ATTN), ln_b.reshape(1, ATTN),
      W_bdy, b_bdy.reshape(1, 1), Wq, Wk, Wv)


NPAD = 20480
LEVELS = 5
_IMAX = 2147483647


def _knn_body(q8_ref, c8t_ref, qc2_ref, cc2_ref, out_ref):
    B = q8_ref.shape[0]
    s = jnp.dot(q8_ref[...], c8t_ref[...], preferred_element_type=jnp.int32)
    d = qc2_ref[...] + (cc2_ref[...] - 2 * s)
    col = jax.lax.broadcasted_iota(jnp.int32, (B, NPAD), 1)
    imin = jnp.int32(-2147483648)
    rkeys = d * 32768 + col + imin
    nslice = NPAD // 128

    top = [rkeys[:, k * 128:(k + 1) * 128] for k in range(LEVELS)]
    for j in range(LEVELS - 1):
        for i in range(LEVELS - 1 - j):
            lo = jnp.minimum(top[i], top[i + 1])
            top[i + 1] = jnp.maximum(top[i], top[i + 1])
            top[i] = lo
    for k in range(LEVELS, nslice):
        v = rkeys[:, k * 128:(k + 1) * 128]
        for i in range(LEVELS):
            lo = jnp.minimum(top[i], v)
            v = jnp.maximum(top[i], v)
            top[i] = lo
    cand = jnp.concatenate(top, axis=1)
    levels = top

    picked = []
    prev1 = jnp.zeros((B, 1), jnp.int32)
    for _ in range(K_SEQ):
        w = jnp.min(cand - prev1, axis=1, keepdims=True)
        picked.append(w + imin + prev1)
        prev1 = w + imin + prev1 + 1
    fast = jnp.concatenate(picked, axis=1)

    k27_rot = picked[-1] + imin
    suspect = jnp.any(levels[-1] <= k27_rot)

    @pl.when(jnp.logical_not(suspect))
    def _():
        out_ref[...] = fast

    @pl.when(suspect)
    def _():
        slow = []
        p1 = jnp.zeros((B, 1), jnp.int32)
        for _ in range(K_SEQ):
            w = jnp.min(rkeys - p1, axis=1, keepdims=True)
            slow.append(w + imin + p1)
            p1 = w + imin + p1 + 1
        out_ref[...] = jnp.concatenate(slow, axis=1)


def _knn_pallas(coords):
    B = 160
    q8 = coords.astype(jnp.int8)
    c8t = jnp.concatenate([q8.T, jnp.zeros((3, NPAD - M), jnp.int8)], axis=1)
    c2 = jnp.sum(coords * coords, axis=1)
    cc2 = jnp.concatenate([c2, jnp.full((NPAD - M,), 30000, jnp.int32)])
    keys27 = pl.pallas_call(
        _knn_body,
        grid=(M // B,),
        in_specs=[
            pl.BlockSpec((B, 3), lambda i: (i, 0)),
            pl.BlockSpec((3, NPAD), lambda i: (0, 0)),
            pl.BlockSpec((B, 1), lambda i: (i, 0)),
            pl.BlockSpec((1, NPAD), lambda i: (0, 0)),
        ],
        out_specs=pl.BlockSpec((B, K_SEQ), lambda i: (i, 0)),
        out_shape=jax.ShapeDtypeStruct((M, K_SEQ), jnp.int32),
    )(q8, c8t, c2[:, None], cc2[None, :])
    return keys27


def kernel(sp_structure, geo_feat_M, sem_feat_M, W_geo, ln_g, ln_b, W_bdy,
           b_bdy, Wq, Wk, Wv, pos_emb, W_out, b_out, W_cls, b_cls):
    Q_geo, bdy_logits, Q_proj, KW, VW = _dense_precompute(
        geo_feat_M, sem_feat_M, W_geo, ln_g, ln_b, W_bdy, b_bdy, Wq, Wk, Wv)

    coords = sp_structure[:, 1:]
    keys27 = _knn_pallas(coords)
    neighbor_idx = keys27 & 32767
    ndist = keys27 >> 15
    valid_mask = ndist <= 3

    tbl = jnp.concatenate([KW, VW, coords.astype(jnp.float32),
                           jnp.zeros((M, 1), jnp.float32)], axis=1)
    from jax.experimental.compute_on import compute_on

    @compute_on("tpu_sparsecore")
    @jax.jit
    def _sc_take(t, i):
        return jnp.take(t, i, axis=0, mode='clip')

    g = _sc_take(tbl, neighbor_idx.reshape(-1))
    g = g.reshape(M, K_SEQ, 132)
    K_g = g[..., :ATTN]
    V_proj = g[..., ATTN:2 * ATTN]
    nc = g[..., 2 * ATTN:2 * ATTN + 3].astype(jnp.int32)
    rel = nc - coords[:, None, :]
    rel_int = jnp.clip(rel + R, 0, 2 * R)
    pos_indices = rel_int[:, :, 0] * DIAM ** 2 + rel_int[:, :, 1] * DIAM + rel_int[:, :, 2]

    qpos = Q_proj @ pos_emb.T
    ph = jax.nn.one_hot(pos_indices, DIAM ** 3, dtype=jnp.float32)
    pos_logit = jnp.einsum('mkp,mp->mk', ph, qpos)
    attn_logits = (jnp.einsum('md,mkd->mk', Q_proj, K_g) + pos_logit) / (ATTN ** 0.5)
    attn_logits = jnp.where(valid_mask, attn_logits, -10000.0)
    affinity = jax.nn.softmax(attn_logits, axis=-1)
    refined = jnp.einsum('mk,mkd->md', affinity, V_proj)
    refined = refined + VW
    refined_feat = refined @ W_out + b_out
    logits = refined_feat @ W_cls + b_cls
    return (logits, bdy_logits, affinity[:, None, :], refined_feat,
            neighbor_idx, valid_mask)
